# Initial kernel scaffold; baseline (speedup 1.0000x reference)
import jax
import jax.numpy as jnp
from jax.experimental import pallas as pl


def kernel(user_ids, item_ids, gmf_user_w, gmf_item_w, mlp_user_w, mlp_item_w, fc1_w, fc1_b, fc2_w, fc2_b, mlp_out_w, mlp_out_b, final_w, final_b):
    raise NotImplementedError("write your pallas kernel here")


# speedup vs baseline: 1.9184x; 1.9184x over previous
import functools

import jax
import jax.numpy as jnp
import numpy as np
from jax import lax
from jax.experimental import pallas as pl
from jax.experimental.pallas import tpu as pltpu
from jax.experimental.pallas import tpu_sc as plsc

B = 16384
V = 1000000
NF = 64
NH = NF // 2
NC = 2
NS = 16
NW = NC * NS
BPW = B // NW
CH = BPW // 2

PC = 8192
PGRID = (V + PC - 1) // PC


def _pack_body(gu_ref, mu_ref, gi_ref, mi_ref, p_ref, o_ref, xcat):
    xcat[0:NF] = gu_ref[...].astype(jnp.bfloat16)
    xcat[NF:96] = mu_ref[...].astype(jnp.bfloat16)
    xcat[96:160] = gi_ref[...].astype(jnp.bfloat16)
    xcat[160:192] = mi_ref[...].astype(jnp.bfloat16)
    lohi = lax.dot_general(xcat[...], p_ref[...], (((0,), (0,)), ((), ())),
                           preferred_element_type=jnp.float32)
    ul = lax.bitcast_convert_type(lohi[:, 0:128], jnp.uint32) >> 16
    uh = (lax.bitcast_convert_type(lohi[:, 128:256], jnp.uint32) >> 16) << 16
    o_ref[...] = lax.bitcast_convert_type(ul | uh, jnp.float32)


def _placements():
    p = np.zeros((192, 256), np.float32)
    a16, a32 = np.arange(16), np.arange(32)
    p[a32, a32] = 1
    p[32 + a32, 128 + a32] = 1
    p[64 + a16, 32 + a16] = 1
    p[80 + a16, 128 + 32 + a16] = 1
    p[96 + a32, 48 + a32] = 1
    p[128 + a32, 128 + 48 + a32] = 1
    p[160 + a16, 80 + a16] = 1
    p[176 + a16, 128 + 80 + a16] = 1
    return jnp.asarray(p, jnp.bfloat16)


def _pack(gut, mut, git, mit, placements):
    return pl.pallas_call(
        _pack_body,
        grid=(PGRID,),
        in_specs=[
            pl.BlockSpec((NF, PC), lambda i: (0, i)),
            pl.BlockSpec((NH, PC), lambda i: (0, i)),
            pl.BlockSpec((NF, PC), lambda i: (0, i)),
            pl.BlockSpec((NH, PC), lambda i: (0, i)),
            pl.BlockSpec((192, 256), lambda i: (0, 0)),
        ],
        out_specs=pl.BlockSpec((PC, 128), lambda i: (i, 0)),
        out_shape=jax.ShapeDtypeStruct((V, 128), jnp.float32),
        scratch_shapes=[pltpu.VMEM((192, PC), jnp.bfloat16)],
        compiler_params=pltpu.CompilerParams(
            dimension_semantics=("parallel",),
            fuse_transposed_lhs_in_matmul=True),
    )(gut, mut, git, mit, placements)


def _sc_gather(ids, tab):
    mesh = plsc.VectorSubcoreMesh(core_axis_name="c", subcore_axis_name="s")

    @functools.partial(
        pl.kernel,
        mesh=mesh,
        out_type=jax.ShapeDtypeStruct((B, 128), jnp.float32),
        scratch_types=[
            pltpu.VMEM((CH,), jnp.int32),
            pltpu.VMEM((CH,), jnp.int32),
            pltpu.VMEM((CH, 128), jnp.float32),
            pltpu.VMEM((CH, 128), jnp.float32),
            pltpu.SemaphoreType.DMA,
            pltpu.SemaphoreType.DMA,
        ],
    )
    def k(ids_hbm, tab_hbm, T, idx0, idx1, buf0, buf1, gsem, wsem):
        wid = lax.axis_index("s") * NC + lax.axis_index("c")
        base = wid * BPW
        pltpu.sync_copy(ids_hbm.at[pl.ds(base, CH)], idx0)
        pltpu.sync_copy(ids_hbm.at[pl.ds(base + CH, CH)], idx1)
        c1 = pltpu.async_copy(tab_hbm.at[idx0], buf0, gsem)
        c2 = pltpu.async_copy(tab_hbm.at[idx1], buf1, gsem)
        c1.wait()
        c2.wait()
        w1 = pltpu.async_copy(buf0, T.at[pl.ds(base, CH)], wsem)
        w2 = pltpu.async_copy(buf1, T.at[pl.ds(base + CH, CH)], wsem)
        w1.wait()
        w2.wait()

    return k(ids, tab)


def _unpack(words):
    u = lax.bitcast_convert_type(words, jnp.uint32)
    lo = lax.bitcast_convert_type(u << 16, jnp.float32)
    hi = lax.bitcast_convert_type((u >> 16) << 16, jnp.float32)
    return lo, hi


def _dense_body(tu_ref, ti_ref, w1ul, w1uh, w1il, w1ih, b1, w2, b2, w3, scal,
                o_ref):
    tu = tu_ref[...]
    ti = ti_ref[...]
    gul, guh = _unpack(tu[:, 0:32])
    gil, gih = _unpack(ti[:, 48:80])
    mul, muh = _unpack(tu[:, 32:48])
    mil, mih = _unpack(ti[:, 80:96])
    g = jnp.maximum(
        jnp.sum(gul * gil + guh * gih, axis=1, keepdims=True), 0.0)
    h = jnp.dot(mul, w1ul[...], preferred_element_type=jnp.float32)
    h = h + jnp.dot(muh, w1uh[...], preferred_element_type=jnp.float32)
    h = h + jnp.dot(mil, w1il[...], preferred_element_type=jnp.float32)
    h = h + jnp.dot(mih, w1ih[...], preferred_element_type=jnp.float32)
    h = jnp.maximum(h + b1[...], 0.0)
    h = jnp.maximum(
        jnp.dot(h, w2[...], preferred_element_type=jnp.float32) + b2[...], 0.0)
    m = jax.nn.sigmoid(jnp.sum(h * w3[...], axis=1, keepdims=True)
                       + scal[:, 0:1])
    o_ref[...] = jax.nn.sigmoid(
        g * scal[:, 1:2] + m * scal[:, 2:3] + scal[:, 3:4])


BB = 2048


def _dense(tu, ti, w1ul, w1uh, w1il, w1ih, b1, w2, b2, w3, scal):
    return pl.pallas_call(
        _dense_body,
        grid=(B // BB,),
        in_specs=[
            pl.BlockSpec((BB, 128), lambda i: (i, 0)),
            pl.BlockSpec((BB, 128), lambda i: (i, 0)),
            pl.BlockSpec((16, NF), lambda i: (0, 0)),
            pl.BlockSpec((16, NF), lambda i: (0, 0)),
            pl.BlockSpec((16, NF), lambda i: (0, 0)),
            pl.BlockSpec((16, NF), lambda i: (0, 0)),
            pl.BlockSpec((1, NF), lambda i: (0, 0)),
            pl.BlockSpec((NF, NF), lambda i: (0, 0)),
            pl.BlockSpec((1, NF), lambda i: (0, 0)),
            pl.BlockSpec((1, NF), lambda i: (0, 0)),
            pl.BlockSpec((1, 4), lambda i: (0, 0)),
        ],
        out_specs=pl.BlockSpec((BB, 1), lambda i: (i, 0)),
        out_shape=jax.ShapeDtypeStruct((B, 1), jnp.float32),
        compiler_params=pltpu.CompilerParams(
            dimension_semantics=("parallel",)),
    )(tu, ti, w1ul, w1uh, w1il, w1ih, b1, w2, b2, w3, scal)


def kernel(user_ids, item_ids, gmf_user_w, gmf_item_w, mlp_user_w, mlp_item_w,
           fc1_w, fc1_b, fc2_w, fc2_b, mlp_out_w, mlp_out_b, final_w, final_b):
    uid = user_ids.astype(jnp.int32)
    iid = item_ids.astype(jnp.int32)
    tab = _pack(gmf_user_w.T, mlp_user_w.T, gmf_item_w.T, mlp_item_w.T,
                _placements())
    TU = _sc_gather(uid, tab)
    TI = _sc_gather(iid, tab)
    w1u = fc1_w[:, :NH].T
    w1i = fc1_w[:, NH:].T
    w1ul = w1u[0:16]
    w1uh = w1u[16:32]
    w1il = w1i[0:16]
    w1ih = w1i[16:32]
    w2 = fc2_w.T
    b1 = fc1_b.reshape(1, NF)
    b2 = fc2_b.reshape(1, NF)
    w3 = mlp_out_w.reshape(1, NF)
    scal = jnp.concatenate(
        [mlp_out_b.reshape(1, 1), final_w.reshape(1, 2),
         final_b.reshape(1, 1)], axis=1)
    return _dense(TU, TI, w1ul, w1uh, w1il, w1ih, b1, w2, b2, w3, scal)
